# Initial kernel scaffold; baseline (speedup 1.0000x reference)
import jax
import jax.numpy as jnp
from jax.experimental import pallas as pl


def kernel(inputs, weight):
    raise NotImplementedError("write your pallas kernel here")


# speedup vs baseline: 1.1496x; 1.1496x over previous
import functools

import jax
import jax.numpy as jnp
from jax import lax
from jax.experimental import pallas as pl
from jax.experimental.pallas import tpu as pltpu
from jax.experimental.pallas import tpu_sc as plsc

NUM_K = 8192
DIM = 32
N_TOTAL = 16384
BN = 256
KC = 2048
NC = 2
NS = 16
NW = NC * NS
ROWS_W = N_TOTAL // NW
HSLICE = NUM_K // NS


def _argmin_body(x_ref, w_ref, idx_ref):
    x = x_ref[...]
    xsq = jnp.sum(x * x, axis=1, keepdims=True)
    best_val = None
    best_idx = None
    for c in range(NUM_K // KC):
        wc = w_ref[pl.ds(c * KC, KC), :]
        wsq = jnp.sum(wc * wc, axis=1)
        mm = lax.dot_general(
            x.astype(jnp.bfloat16), wc.astype(jnp.bfloat16),
            (((1,), (1,)), ((), ())),
            preferred_element_type=jnp.float32)
        dist = (xsq + wsq[None, :]) - 2.0 * mm
        m = jnp.min(dist, axis=1)
        iota = lax.broadcasted_iota(jnp.int32, (BN, KC), 1) + c * KC
        am = jnp.min(
            jnp.where(dist == m[:, None], iota, jnp.int32(2 ** 30)), axis=1)
        if c == 0:
            best_val, best_idx = m, am
        else:
            best_idx = jnp.where(m < best_val, am, best_idx)
            best_val = jnp.minimum(best_val, m)
        if c % 2 == 1:
            u = lax.bitcast_convert_type(best_val, jnp.uint32)
            u = (u + jnp.uint32(0x7FFF) + ((u >> 16) & jnp.uint32(1))) \
                & jnp.uint32(0xFFFF0000)
            best_val = lax.bitcast_convert_type(u, jnp.float32)
    idx_ref[...] = best_idx


def _argmin_call(x2d, weight):
    return pl.pallas_call(
        _argmin_body,
        grid=(N_TOTAL // BN,),
        in_specs=[
            pl.BlockSpec((BN, DIM), lambda i: (i, 0)),
            pl.BlockSpec((NUM_K, DIM), lambda i: (0, 0)),
        ],
        out_specs=pl.BlockSpec((BN,), lambda i: (i,)),
        out_shape=jax.ShapeDtypeStruct((N_TOTAL,), jnp.int32),
    )(x2d, weight)


def _sc_body(w_hbm, idx_hbm, q_hbm, hist_hbm,
             idx_v, rows_v, ones_v, buf_v, shared_hist, sem):
    cid = lax.axis_index("c")
    sid = lax.axis_index("s")
    wid = sid * NC + cid
    base = wid * ROWS_W

    zero16 = jnp.zeros((16,), jnp.float32)
    for i in range(HSLICE // 16):
        buf_v[pl.ds(i * 16, 16)] = zero16
    pltpu.sync_copy(buf_v, shared_hist.at[pl.ds(sid * HSLICE, HSLICE)])

    pltpu.sync_copy(idx_hbm.at[pl.ds(base, ROWS_W)], idx_v)
    pltpu.async_copy(w_hbm.at[idx_v], rows_v, sem).wait()
    pltpu.sync_copy(rows_v, q_hbm.at[pl.ds(base, ROWS_W)])

    one16 = jnp.full((16,), 1.0, jnp.float32)
    for i in range(ROWS_W // 16):
        ones_v[pl.ds(i * 16, 16)] = one16
    plsc.subcore_barrier()
    pltpu.sync_copy(ones_v, shared_hist.at[idx_v], add=True)
    plsc.subcore_barrier()

    pltpu.sync_copy(shared_hist.at[pl.ds(sid * HSLICE, HSLICE)], buf_v)
    pltpu.sync_copy(buf_v, hist_hbm.at[pl.ds(cid * NUM_K + sid * HSLICE, HSLICE)])


@jax.jit
def _sc_call(weight, idx):
    mesh = plsc.VectorSubcoreMesh(core_axis_name="c", subcore_axis_name="s")
    return pl.kernel(
        _sc_body,
        out_type=(
            jax.ShapeDtypeStruct((N_TOTAL, DIM), jnp.float32),
            jax.ShapeDtypeStruct((NC * NUM_K,), jnp.float32),
        ),
        mesh=mesh,
        scratch_types=[
            pltpu.VMEM((ROWS_W,), jnp.int32),
            pltpu.VMEM((ROWS_W, DIM), jnp.float32),
            pltpu.VMEM((ROWS_W,), jnp.float32),
            pltpu.VMEM((HSLICE,), jnp.float32),
            pltpu.VMEM_SHARED((NUM_K,), jnp.float32),
            pltpu.SemaphoreType.DMA,
        ],
        compiler_params=pltpu.CompilerParams(use_tc_tiling_on_sc=False),
    )(weight, idx)


N_TILES_C = 16
BN_C = N_TOTAL // N_TILES_C


BF_C = N_TOTAL * DIM // N_TILES_C


def _finish_body(x_ref, q_ref, hist_ref, qst_ref, loss_ref, perp_ref, acc_ref):
    i = pl.program_id(0)
    x = x_ref[...]
    q = q_ref[...]
    d = q - x
    qst_ref[...] = x + d
    part = jnp.sum(d * d)

    @pl.when(i == 0)
    def _():
        acc_ref[0] = part

    @pl.when(i > 0)
    def _():
        acc_ref[0] = acc_ref[0] + part

    @pl.when(i == N_TILES_C - 1)
    def _():
        m = acc_ref[0] * (1.0 / (N_TOTAL * DIM))
        loss_ref[...] = jnp.broadcast_to(m + 0.25 * m, (1, 1))
        hist = hist_ref[...]
        p = (hist[0:NUM_K] + hist[NUM_K:2 * NUM_K]) * (1.0 / N_TOTAL)
        ent = jnp.sum(p * jnp.log(p + 1e-10))
        perp_ref[...] = jnp.broadcast_to(jnp.exp(-ent), (1, 1))


def _finish_call(x_flat, q_flat, hist_flat):
    return pl.pallas_call(
        _finish_body,
        grid=(N_TILES_C,),
        in_specs=[
            pl.BlockSpec((BF_C,), lambda i: (i,)),
            pl.BlockSpec((BF_C,), lambda i: (i,)),
            pl.BlockSpec((NC * NUM_K,), lambda i: (0,)),
        ],
        out_specs=[
            pl.BlockSpec((BF_C,), lambda i: (i,)),
            pl.BlockSpec((1, 1), lambda i: (0, 0)),
            pl.BlockSpec((1, 1), lambda i: (0, 0)),
        ],
        out_shape=[
            jax.ShapeDtypeStruct((N_TOTAL * DIM,), jnp.float32),
            jax.ShapeDtypeStruct((1, 1), jnp.float32),
            jax.ShapeDtypeStruct((1, 1), jnp.float32),
        ],
        scratch_shapes=[pltpu.SMEM((1,), jnp.float32)],
    )(x_flat, q_flat, hist_flat)


def kernel(inputs, weight):
    x2d = inputs.reshape(N_TOTAL, DIM)
    idx = _argmin_call(x2d, weight)
    q2d, hist_flat = _sc_call(weight, idx)
    qst_flat, loss, perp = _finish_call(
        inputs.reshape(-1), q2d.reshape(-1), hist_flat)
    return (qst_flat.reshape(inputs.shape), loss[0, 0], perp[0, 0])
